# baseline (speedup 1.0000x reference)
import functools
import math

import jax
import jax.numpy as jnp
from jax import lax
from jax.experimental import pallas as pl
from jax.experimental.pallas import tpu as pltpu
from jax.experimental.pallas import tpu_sc as plsc

NC = 2
NS = 16
NW = NC * NS
G = 128
NBUF = 4
PREP_B = 1024


def _prep_body(emb, scale, tab_ref, out_ref):
    out_ref[:, :emb] = tab_ref[...] * jnp.float32(scale)


def _emb_body(n_tokens, tok_hbm, table_hbm, out_hbm,
              idx_full, rows, gsems, ssems):
    steps = n_tokens // NW // G
    wid = lax.axis_index("s") * NC + lax.axis_index("c")
    idx_row0 = wid * steps
    row0 = wid * steps * G

    pltpu.sync_copy(tok_hbm.at[pl.ds(idx_row0, steps)], idx_full)

    def start_gather(s, b):
        pltpu.async_copy(table_hbm.at[idx_full.at[s]], rows[b], gsems[b])

    def wait_gather(b):
        pltpu.make_async_copy(table_hbm.at[idx_full.at[0]], rows[b],
                              gsems[b]).wait()

    def start_store(s, b):
        pltpu.async_copy(rows[b], out_hbm.at[pl.ds(row0 + s * G, G)], ssems[b])

    def wait_store(b):
        pltpu.make_async_copy(rows[b], out_hbm.at[pl.ds(row0, G)],
                              ssems[b]).wait()

    for b in range(2):
        start_gather(b, b)

    def quad(q, carry):
        for k in range(NBUF):
            s = NBUF * q + k
            b = k
            wait_gather(b)
            start_store(s, b)
            b2 = (k + 2) % NBUF

            @pl.when(s >= 2)
            def _():
                wait_store(b2)

            start_gather(jnp.minimum(s + 2, steps - 1), b2)
        return carry

    lax.fori_loop(0, steps // NBUF, quad, 0)
    for b in ((steps - 2) % NBUF, (steps - 1) % NBUF):
        wait_store(b)
    for b in (steps % NBUF, (steps + 1) % NBUF):
        wait_gather(b)


def kernel(tokens, table):
    bsz, seq = tokens.shape
    vocab, emb = table.shape
    n_tokens = bsz * seq

    tab128 = pl.pallas_call(
        functools.partial(_prep_body, emb, math.sqrt(emb)),
        grid=((vocab + PREP_B - 1) // PREP_B,),
        in_specs=[pl.BlockSpec((PREP_B, emb), lambda i: (i, 0))],
        out_specs=pl.BlockSpec((PREP_B, 128), lambda i: (i, 0)),
        out_shape=jax.ShapeDtypeStruct((vocab, 128), jnp.float32),
    )(table)

    tok = tokens.reshape(n_tokens // G, G).astype(jnp.int32)
    mesh = plsc.VectorSubcoreMesh(core_axis_name="c", subcore_axis_name="s",
                                  num_cores=NC, num_subcores=NS)
    steps = n_tokens // NW // G
    run = pl.kernel(
        functools.partial(_emb_body, n_tokens),
        out_type=jax.ShapeDtypeStruct((n_tokens, 128), jnp.float32),
        mesh=mesh,
        scratch_types=[
            pltpu.VMEM((steps, G), jnp.int32),
            [pltpu.VMEM((G, 128), jnp.float32) for _ in range(NBUF)],
            [pltpu.SemaphoreType.DMA for _ in range(NBUF)],
            [pltpu.SemaphoreType.DMA for _ in range(NBUF)],
        ],
        compiler_params=pltpu.CompilerParams(use_tc_tiling_on_sc=True),
    )
    out = run(tok, tab128)
    return out[:, :emb].reshape(bsz, seq, emb)


# speedup vs baseline: 2.0134x; 2.0134x over previous
import functools
import math

import jax
import jax.numpy as jnp
from jax import lax
from jax.experimental import pallas as pl
from jax.experimental.pallas import tpu as pltpu
from jax.experimental.pallas import tpu_sc as plsc

NC = 2
NS = 16
NW = NC * NS
G = 128
NBUF = 4
PREP_B = 8192


def _prep_body(emb, scale, tabt_ref, out_ref):
    x = tabt_ref[...]
    out_ref[:, :emb] = x.T * jnp.float32(scale)


def _emb_body(n_tokens, tok_hbm, table_hbm, out_hbm,
              idx_full, rows, gsems, ssems):
    steps = n_tokens // NW // G
    wid = lax.axis_index("s") * NC + lax.axis_index("c")
    idx_row0 = wid * steps
    row0 = wid * steps * G

    pltpu.sync_copy(tok_hbm.at[pl.ds(idx_row0, steps)], idx_full)

    def start_gather(s, b):
        pltpu.async_copy(table_hbm.at[idx_full.at[s]], rows[b], gsems[b])

    def wait_gather(b):
        pltpu.make_async_copy(table_hbm.at[idx_full.at[0]], rows[b],
                              gsems[b]).wait()

    def start_store(s, b):
        pltpu.async_copy(rows[b], out_hbm.at[pl.ds(row0 + s * G, G)], ssems[b])

    def wait_store(b):
        pltpu.make_async_copy(rows[b], out_hbm.at[pl.ds(row0, G)],
                              ssems[b]).wait()

    for b in range(2):
        start_gather(b, b)

    def quad(q, carry):
        for k in range(NBUF):
            s = NBUF * q + k
            b = k
            wait_gather(b)
            start_store(s, b)
            b2 = (k + 2) % NBUF

            @pl.when(s >= 2)
            def _():
                wait_store(b2)

            start_gather(jnp.minimum(s + 2, steps - 1), b2)
        return carry

    lax.fori_loop(0, steps // NBUF, quad, 0)
    for b in ((steps - 2) % NBUF, (steps - 1) % NBUF):
        wait_store(b)
    for b in (steps % NBUF, (steps + 1) % NBUF):
        wait_gather(b)


def kernel(tokens, table):
    bsz, seq = tokens.shape
    vocab, emb = table.shape
    n_tokens = bsz * seq

    tab128 = pl.pallas_call(
        functools.partial(_prep_body, emb, math.sqrt(emb)),
        grid=((vocab + PREP_B - 1) // PREP_B,),
        in_specs=[pl.BlockSpec((emb, PREP_B), lambda i: (0, i))],
        out_specs=pl.BlockSpec((PREP_B, 128), lambda i: (i, 0)),
        out_shape=jax.ShapeDtypeStruct((vocab, 128), jnp.float32),
    )(table.T)

    tok = tokens.reshape(n_tokens // G, G).astype(jnp.int32)
    mesh = plsc.VectorSubcoreMesh(core_axis_name="c", subcore_axis_name="s",
                                  num_cores=NC, num_subcores=NS)
    steps = n_tokens // NW // G
    run = pl.kernel(
        functools.partial(_emb_body, n_tokens),
        out_type=jax.ShapeDtypeStruct((n_tokens, 128), jnp.float32),
        mesh=mesh,
        scratch_types=[
            pltpu.VMEM((steps, G), jnp.int32),
            [pltpu.VMEM((G, 128), jnp.float32) for _ in range(NBUF)],
            [pltpu.SemaphoreType.DMA for _ in range(NBUF)],
            [pltpu.SemaphoreType.DMA for _ in range(NBUF)],
        ],
        compiler_params=pltpu.CompilerParams(use_tc_tiling_on_sc=True),
    )
    out = run(tok, tab128)
    return out[:, :emb].reshape(bsz, seq, emb)


# speedup vs baseline: 2.0849x; 1.0355x over previous
import functools
import math

import jax
import jax.numpy as jnp
from jax import lax
from jax.experimental import pallas as pl
from jax.experimental.pallas import tpu as pltpu
from jax.experimental.pallas import tpu_sc as plsc

NC = 2
NS = 16
NW = NC * NS
G = 128
NBUF = 4
PREP_B = 32768


def _prep_body(emb, scale, tabt_ref, out_ref):
    x = tabt_ref[...]
    out_ref[:, :emb] = x.T * jnp.float32(scale)


def _emb_body(n_tokens, tok_hbm, table_hbm, out_hbm,
              idx_full, rows, gsems, ssems):
    steps = n_tokens // NW // G
    wid = lax.axis_index("s") * NC + lax.axis_index("c")
    idx_row0 = wid * steps
    row0 = wid * steps * G

    pltpu.sync_copy(tok_hbm.at[pl.ds(idx_row0, steps)], idx_full)

    def start_gather(s, b):
        pltpu.async_copy(table_hbm.at[idx_full.at[s]], rows[b], gsems[b])

    def wait_gather(b):
        pltpu.make_async_copy(table_hbm.at[idx_full.at[0]], rows[b],
                              gsems[b]).wait()

    def start_store(s, b):
        pltpu.async_copy(rows[b], out_hbm.at[pl.ds(row0 + s * G, G)], ssems[b])

    def wait_store(b):
        pltpu.make_async_copy(rows[b], out_hbm.at[pl.ds(row0, G)],
                              ssems[b]).wait()

    for b in range(2):
        start_gather(b, b)

    def quad(q, carry):
        for k in range(NBUF):
            s = NBUF * q + k
            b = k
            wait_gather(b)
            start_store(s, b)
            b2 = (k + 2) % NBUF

            @pl.when(s >= 2)
            def _():
                wait_store(b2)

            start_gather(jnp.minimum(s + 2, steps - 1), b2)
        return carry

    lax.fori_loop(0, steps // NBUF, quad, 0)
    for b in ((steps - 2) % NBUF, (steps - 1) % NBUF):
        wait_store(b)
    for b in (steps % NBUF, (steps + 1) % NBUF):
        wait_gather(b)


def kernel(tokens, table):
    bsz, seq = tokens.shape
    vocab, emb = table.shape
    n_tokens = bsz * seq

    tab128 = pl.pallas_call(
        functools.partial(_prep_body, emb, math.sqrt(emb)),
        grid=((vocab + PREP_B - 1) // PREP_B,),
        in_specs=[pl.BlockSpec((emb, PREP_B), lambda i: (0, i))],
        out_specs=pl.BlockSpec((PREP_B, 128), lambda i: (i, 0)),
        out_shape=jax.ShapeDtypeStruct((vocab, 128), jnp.float32),
    )(table.T)

    tok = tokens.reshape(n_tokens // G, G).astype(jnp.int32)
    mesh = plsc.VectorSubcoreMesh(core_axis_name="c", subcore_axis_name="s",
                                  num_cores=NC, num_subcores=NS)
    steps = n_tokens // NW // G
    run = pl.kernel(
        functools.partial(_emb_body, n_tokens),
        out_type=jax.ShapeDtypeStruct((n_tokens, 128), jnp.float32),
        mesh=mesh,
        scratch_types=[
            pltpu.VMEM((steps, G), jnp.int32),
            [pltpu.VMEM((G, 128), jnp.float32) for _ in range(NBUF)],
            [pltpu.SemaphoreType.DMA for _ in range(NBUF)],
            [pltpu.SemaphoreType.DMA for _ in range(NBUF)],
        ],
        compiler_params=pltpu.CompilerParams(use_tc_tiling_on_sc=True),
    )
    out = run(tok, tab128)
    return out[:, :emb].reshape(bsz, seq, emb)
